# Initial kernel scaffold; baseline (speedup 1.0000x reference)
import jax
import jax.numpy as jnp
from jax.experimental import pallas as pl


def kernel(x, edge_index, W1, b1, W2, b2, Wd, bd):
    raise NotImplementedError("write your pallas kernel here")


# speedup vs baseline: 16.3896x; 16.3896x over previous
import functools

import jax
import jax.numpy as jnp
from jax import lax
from jax.experimental import pallas as pl
from jax.experimental.pallas import tpu as pltpu
from jax.experimental.pallas import tpu_sc as plsc

NC = 2
NS = 16
NW = NC * NS
CHUNK = 128
BLK = 512
D = 128
H = 16


def _sc_mesh():
    return plsc.VectorSubcoreMesh(core_axis_name="c", subcore_axis_name="s",
                                  num_cores=NC, num_subcores=NS)


@functools.lru_cache(maxsize=None)
def _make_prop(NP, NCH, F):
    rps = NP // NS
    assert rps % 128 == 0

    @functools.partial(
        pl.kernel,
        out_type=jax.ShapeDtypeStruct((NC, NP, F), jnp.float32),
        mesh=_sc_mesh(),
        compiler_params=pltpu.CompilerParams(
            use_tc_tiling_on_sc=(False if F % 128 else None)),
        scratch_types=[
            pltpu.VMEM((NCH, CHUNK), jnp.int32),
            pltpu.VMEM((NCH, CHUNK), jnp.int32),
            pltpu.VMEM((CHUNK, F), jnp.float32),
            pltpu.VMEM((128, F), jnp.float32),
            pltpu.VMEM_SHARED((NP, F), jnp.float32),
            pltpu.SemaphoreType.DMA,
        ],
    )
    def prop(y_hbm, src_hbm, dst_hbm, out_hbm, idx_s, idx_d, rows, zbuf, acc, sem):
        c = lax.axis_index("c")
        s = lax.axis_index("s")
        wid = c * NS + s
        pltpu.sync_copy(src_hbm.at[wid], idx_s)
        pltpu.sync_copy(dst_hbm.at[wid], idx_d)

        def zb(i, carry):
            for k in range(F // 16):
                zbuf[i, pl.ds(k * 16, 16)] = jnp.zeros((16,), jnp.float32)
            return carry

        lax.fori_loop(0, 128, zb, 0)
        for k in range(rps // 128):
            pltpu.sync_copy(zbuf, acc.at[pl.ds(s * rps + k * 128, 128)])
        plsc.subcore_barrier()

        def step(j, carry):
            pltpu.async_copy(y_hbm.at[idx_s.at[j]], rows, sem).wait()
            pltpu.sync_copy(rows, acc.at[idx_d.at[j]], add=True)
            return carry

        lax.fori_loop(0, NCH, step, 0)
        plsc.subcore_barrier()
        pltpu.sync_copy(acc.at[pl.ds(s * rps, rps)],
                        out_hbm.at[c, pl.ds(s * rps, rps)])

    return prop


@functools.lru_cache(maxsize=None)
def _make_deg(NP, NCH):
    rps = NP // NS

    @functools.partial(
        pl.kernel,
        out_type=jax.ShapeDtypeStruct((NC, NP), jnp.float32),
        mesh=_sc_mesh(),
        compiler_params=pltpu.CompilerParams(use_tc_tiling_on_sc=False),
        scratch_types=[
            pltpu.VMEM((NCH, CHUNK), jnp.int32),
            pltpu.VMEM((CHUNK,), jnp.float32),
            pltpu.VMEM((rps,), jnp.float32),
            pltpu.VMEM_SHARED((NP,), jnp.float32),
        ],
    )
    def deg(dst_hbm, out_hbm, idx_d, ones, zbuf, acc):
        c = lax.axis_index("c")
        s = lax.axis_index("s")
        wid = c * NS + s
        pltpu.sync_copy(dst_hbm.at[wid], idx_d)
        for i in range(CHUNK // 16):
            ones[pl.ds(i * 16, 16)] = jnp.ones((16,), jnp.float32)

        def zb(i, carry):
            zbuf[pl.ds(i * 16, 16)] = jnp.zeros((16,), jnp.float32)
            return carry

        lax.fori_loop(0, rps // 16, zb, 0)
        pltpu.sync_copy(zbuf, acc.at[pl.ds(s * rps, rps)])
        plsc.subcore_barrier()

        def step(j, carry):
            pltpu.sync_copy(ones, acc.at[idx_d.at[j]], add=True)
            return carry

        lax.fori_loop(0, NCH, step, 0)
        plsc.subcore_barrier()
        pltpu.sync_copy(acc.at[pl.ds(s * rps, rps)],
                        out_hbm.at[c, pl.ds(s * rps, rps)])

    return deg


def _dinv(deg_ref):
    d = deg_ref[:, 0:1] + deg_ref[:, 1:2] + 1.0
    return lax.rsqrt(jnp.maximum(d, 1.0))


def _tc0_body(x_ref, deg_ref, ya_ref, yb_ref):
    y = x_ref[...] * _dinv(deg_ref)
    ya_ref[...] = y[:, : D // 2]
    yb_ref[...] = y[:, D // 2:]


def _tc0(x_pad, degt, NP):
    return pl.pallas_call(
        _tc0_body,
        grid=(NP // BLK,),
        in_specs=[
            pl.BlockSpec((BLK, D), lambda i: (i, 0)),
            pl.BlockSpec((BLK, NC), lambda i: (i, 0)),
        ],
        out_specs=[pl.BlockSpec((BLK, D // 2), lambda i: (i, 0))] * 2,
        out_shape=[jax.ShapeDtypeStruct((NP, D // 2), jnp.float32)] * 2,
    )(x_pad, degt)


def _make_tc1_body(N):
    def body(pa_ref, pb_ref, ya_ref, yb_ref, deg_ref, b_ref, wa_ref, wb_ref,
             y1_ref):
        i = pl.program_id(0)
        dinv = _dinv(deg_ref)
        agg_a = (pa_ref[0] + pa_ref[1] + ya_ref[...]) * dinv
        agg_b = (pb_ref[0] + pb_ref[1] + yb_ref[...]) * dinv
        aw = jnp.dot(agg_a, wa_ref[...]) + jnp.dot(agg_b, wb_ref[...])
        h1 = jnp.maximum(aw + b_ref[...], 0.0)
        y1 = h1 * dinv
        row = lax.broadcasted_iota(jnp.int32, (BLK, 1), 0) + i * BLK
        y1_ref[...] = jnp.where(row < N, y1, 0.0)

    return body


def _tc1(p1a, p1b, y0a, y0b, degt, b1, w1, NP, N):
    return pl.pallas_call(
        _make_tc1_body(N),
        grid=(NP // BLK,),
        in_specs=[
            pl.BlockSpec((NC, BLK, D // 2), lambda i: (0, i, 0)),
            pl.BlockSpec((NC, BLK, D // 2), lambda i: (0, i, 0)),
            pl.BlockSpec((BLK, D // 2), lambda i: (i, 0)),
            pl.BlockSpec((BLK, D // 2), lambda i: (i, 0)),
            pl.BlockSpec((BLK, NC), lambda i: (i, 0)),
            pl.BlockSpec((1, H), lambda i: (0, 0)),
            pl.BlockSpec((D // 2, H), lambda i: (0, 0)),
            pl.BlockSpec((D // 2, H), lambda i: (0, 0)),
        ],
        out_specs=pl.BlockSpec((BLK, H), lambda i: (i, 0)),
        out_shape=jax.ShapeDtypeStruct((NP, H), jnp.float32),
    )(p1a, p1b, y0a, y0b, degt, b1, w1[: D // 2], w1[D // 2:])


def _tc2_body(p_ref, y1_ref, deg_ref, b_ref, w_ref, wd_ref, bd_ref, o_ref):
    dinv = _dinv(deg_ref)
    agg = (p_ref[0] + p_ref[1] + y1_ref[...]) * dinv
    h2 = jnp.maximum(jnp.dot(agg, w_ref[...]) + b_ref[...], 0.0)
    o_ref[...] = jnp.dot(h2, wd_ref[...]) + bd_ref[...]


def _tc2(p2, y1, degt, b2, w2, wd, bd, NP):
    return pl.pallas_call(
        _tc2_body,
        grid=(NP // BLK,),
        in_specs=[
            pl.BlockSpec((NC, BLK, H), lambda i: (0, i, 0)),
            pl.BlockSpec((BLK, H), lambda i: (i, 0)),
            pl.BlockSpec((BLK, NC), lambda i: (i, 0)),
            pl.BlockSpec((1, H), lambda i: (0, 0)),
            pl.BlockSpec((H, H), lambda i: (0, 0)),
            pl.BlockSpec((H, 1), lambda i: (0, 0)),
            pl.BlockSpec((1, 1), lambda i: (0, 0)),
        ],
        out_specs=pl.BlockSpec((BLK, 1), lambda i: (i, 0)),
        out_shape=jax.ShapeDtypeStruct((NP, 1), jnp.float32),
    )(p2, y1, degt, b2, w2, wd, bd)


def kernel(x, edge_index, W1, b1, W2, b2, Wd, bd):
    N = x.shape[0]
    E = edge_index.shape[1]
    NP = -(-(N + 8) // (NS * 128)) * NS * 128
    EP = -(-E // (NW * CHUNK)) * NW * CHUNK
    NCH = EP // (NW * CHUNK)

    src = edge_index[0].astype(jnp.int32)
    dst = edge_index[1].astype(jnp.int32)
    pad = jnp.full((EP - E,), N, jnp.int32)
    srcp = jnp.concatenate([src, pad]).reshape(NW, NCH, CHUNK)
    dstp = jnp.concatenate([dst, pad]).reshape(NW, NCH, CHUNK)
    x_pad = jnp.zeros((NP, D), jnp.float32).at[:N].set(x)

    degp = _make_deg(NP, NCH)(dstp)
    degt = degp.T

    y0a, y0b = _tc0(x_pad, degt, NP)
    prop64 = _make_prop(NP, NCH, D // 2)
    p1a = prop64(y0a, srcp, dstp)
    p1b = prop64(y0b, srcp, dstp)
    y1 = _tc1(p1a, p1b, y0a, y0b, degt, b1.reshape(1, H), W1, NP, N)
    p2 = _make_prop(NP, NCH, H)(y1, srcp, dstp)
    out = _tc2(p2, y1, degt, b2.reshape(1, H), W2, Wd, bd.reshape(1, 1), NP)
    return out[:N]


# speedup vs baseline: 20.8577x; 1.2726x over previous
import functools

import jax
import jax.numpy as jnp
from jax import lax
from jax.experimental import pallas as pl
from jax.experimental.pallas import tpu as pltpu
from jax.experimental.pallas import tpu_sc as plsc

NC = 2
NS = 16
NW = NC * NS
CHUNK = 128
BLK = 512
D = 128
H = 16


def _sc_mesh():
    return plsc.VectorSubcoreMesh(core_axis_name="c", subcore_axis_name="s",
                                  num_cores=NC, num_subcores=NS)


@functools.lru_cache(maxsize=None)
def _make_prop(NP, NCH, F):
    rps = NP // NS
    assert rps % 128 == 0

    @functools.partial(
        pl.kernel,
        out_type=jax.ShapeDtypeStruct((NC, NP, F), jnp.float32),
        mesh=_sc_mesh(),
        compiler_params=pltpu.CompilerParams(
            use_tc_tiling_on_sc=(False if F % 128 else None)),
        scratch_types=[
            pltpu.VMEM((NCH, CHUNK), jnp.int32),
            pltpu.VMEM((NCH, CHUNK), jnp.int32),
            pltpu.VMEM((CHUNK, F), jnp.float32),
            pltpu.VMEM((CHUNK, F), jnp.float32),
            pltpu.VMEM((128, F), jnp.float32),
            pltpu.VMEM_SHARED((NP, F), jnp.float32),
            pltpu.SemaphoreType.DMA,
            pltpu.SemaphoreType.DMA,
        ],
    )
    def prop(y_hbm, src_hbm, dst_hbm, out_hbm, idx_s, idx_d, rows0, rows1,
             zbuf, acc, sem0, sem1):
        c = lax.axis_index("c")
        s = lax.axis_index("s")
        wid = c * NS + s
        pltpu.sync_copy(src_hbm.at[wid], idx_s)
        pltpu.sync_copy(dst_hbm.at[wid], idx_d)

        def zb(i, carry):
            for k in range(F // 16):
                zbuf[i, pl.ds(k * 16, 16)] = jnp.zeros((16,), jnp.float32)
            return carry

        lax.fori_loop(0, 128, zb, 0)
        for k in range(rps // 128):
            pltpu.sync_copy(zbuf, acc.at[pl.ds(s * rps + k * 128, 128)])
        plsc.subcore_barrier()

        pltpu.async_copy(y_hbm.at[idx_s.at[0]], rows0, sem0)

        def pair(i, carry):
            j0 = 2 * i
            pltpu.async_copy(y_hbm.at[idx_s.at[j0 + 1]], rows1, sem1)
            pltpu.make_async_copy(y_hbm.at[idx_s.at[j0]], rows0, sem0).wait()
            pltpu.sync_copy(rows0, acc.at[idx_d.at[j0]], add=True)

            @pl.when(j0 + 2 < NCH)
            def _():
                pltpu.async_copy(y_hbm.at[idx_s.at[j0 + 2]], rows0, sem0)

            pltpu.make_async_copy(y_hbm.at[idx_s.at[j0 + 1]], rows1, sem1).wait()
            pltpu.sync_copy(rows1, acc.at[idx_d.at[j0 + 1]], add=True)
            return carry

        lax.fori_loop(0, NCH // 2, pair, 0)
        if NCH % 2:
            pltpu.make_async_copy(y_hbm.at[idx_s.at[NCH - 1]], rows0, sem0).wait()
            pltpu.sync_copy(rows0, acc.at[idx_d.at[NCH - 1]], add=True)
        plsc.subcore_barrier()
        pltpu.sync_copy(acc.at[pl.ds(s * rps, rps)],
                        out_hbm.at[c, pl.ds(s * rps, rps)])

    return prop


@functools.lru_cache(maxsize=None)
def _make_deg(NP, NCH):
    rps = NP // NS

    @functools.partial(
        pl.kernel,
        out_type=jax.ShapeDtypeStruct((NC, NP), jnp.float32),
        mesh=_sc_mesh(),
        compiler_params=pltpu.CompilerParams(use_tc_tiling_on_sc=False),
        scratch_types=[
            pltpu.VMEM((NCH, CHUNK), jnp.int32),
            pltpu.VMEM((CHUNK,), jnp.float32),
            pltpu.VMEM((rps,), jnp.float32),
            pltpu.VMEM_SHARED((NP,), jnp.float32),
        ],
    )
    def deg(dst_hbm, out_hbm, idx_d, ones, zbuf, acc):
        c = lax.axis_index("c")
        s = lax.axis_index("s")
        wid = c * NS + s
        pltpu.sync_copy(dst_hbm.at[wid], idx_d)
        for i in range(CHUNK // 16):
            ones[pl.ds(i * 16, 16)] = jnp.ones((16,), jnp.float32)

        def zb(i, carry):
            zbuf[pl.ds(i * 16, 16)] = jnp.zeros((16,), jnp.float32)
            return carry

        lax.fori_loop(0, rps // 16, zb, 0)
        pltpu.sync_copy(zbuf, acc.at[pl.ds(s * rps, rps)])
        plsc.subcore_barrier()

        def step(j, carry):
            pltpu.sync_copy(ones, acc.at[idx_d.at[j]], add=True)
            return carry

        lax.fori_loop(0, NCH, step, 0)
        plsc.subcore_barrier()
        pltpu.sync_copy(acc.at[pl.ds(s * rps, rps)],
                        out_hbm.at[c, pl.ds(s * rps, rps)])

    return deg


def _dinv(deg_ref):
    d = deg_ref[:, 0:1] + deg_ref[:, 1:2] + 1.0
    return lax.rsqrt(jnp.maximum(d, 1.0))


def _tc0_body(x_ref, deg_ref, ya_ref, yb_ref):
    y = x_ref[...] * _dinv(deg_ref)
    ya_ref[...] = y[:, : D // 2]
    yb_ref[...] = y[:, D // 2:]


def _tc0(x_pad, degt, NP):
    return pl.pallas_call(
        _tc0_body,
        grid=(NP // BLK,),
        in_specs=[
            pl.BlockSpec((BLK, D), lambda i: (i, 0)),
            pl.BlockSpec((BLK, NC), lambda i: (i, 0)),
        ],
        out_specs=[pl.BlockSpec((BLK, D // 2), lambda i: (i, 0))] * 2,
        out_shape=[jax.ShapeDtypeStruct((NP, D // 2), jnp.float32)] * 2,
    )(x_pad, degt)


def _make_tc1_body(N):
    def body(pa_ref, pb_ref, ya_ref, yb_ref, deg_ref, b_ref, wa_ref, wb_ref,
             y1_ref):
        i = pl.program_id(0)
        dinv = _dinv(deg_ref)
        agg_a = (pa_ref[0] + pa_ref[1] + ya_ref[...]) * dinv
        agg_b = (pb_ref[0] + pb_ref[1] + yb_ref[...]) * dinv
        aw = jnp.dot(agg_a, wa_ref[...]) + jnp.dot(agg_b, wb_ref[...])
        h1 = jnp.maximum(aw + b_ref[...], 0.0)
        y1 = h1 * dinv
        row = lax.broadcasted_iota(jnp.int32, (BLK, 1), 0) + i * BLK
        y1_ref[...] = jnp.where(row < N, y1, 0.0)

    return body


def _tc1(p1a, p1b, y0a, y0b, degt, b1, w1, NP, N):
    return pl.pallas_call(
        _make_tc1_body(N),
        grid=(NP // BLK,),
        in_specs=[
            pl.BlockSpec((NC, BLK, D // 2), lambda i: (0, i, 0)),
            pl.BlockSpec((NC, BLK, D // 2), lambda i: (0, i, 0)),
            pl.BlockSpec((BLK, D // 2), lambda i: (i, 0)),
            pl.BlockSpec((BLK, D // 2), lambda i: (i, 0)),
            pl.BlockSpec((BLK, NC), lambda i: (i, 0)),
            pl.BlockSpec((1, H), lambda i: (0, 0)),
            pl.BlockSpec((D // 2, H), lambda i: (0, 0)),
            pl.BlockSpec((D // 2, H), lambda i: (0, 0)),
        ],
        out_specs=pl.BlockSpec((BLK, H), lambda i: (i, 0)),
        out_shape=jax.ShapeDtypeStruct((NP, H), jnp.float32),
    )(p1a, p1b, y0a, y0b, degt, b1, w1[: D // 2], w1[D // 2:])


def _tc2_body(p_ref, y1_ref, deg_ref, b_ref, w_ref, wd_ref, bd_ref, o_ref):
    dinv = _dinv(deg_ref)
    agg = (p_ref[0] + p_ref[1] + y1_ref[...]) * dinv
    h2 = jnp.maximum(jnp.dot(agg, w_ref[...]) + b_ref[...], 0.0)
    o_ref[...] = jnp.dot(h2, wd_ref[...]) + bd_ref[...]


def _tc2(p2, y1, degt, b2, w2, wd, bd, NP):
    return pl.pallas_call(
        _tc2_body,
        grid=(NP // BLK,),
        in_specs=[
            pl.BlockSpec((NC, BLK, H), lambda i: (0, i, 0)),
            pl.BlockSpec((BLK, H), lambda i: (i, 0)),
            pl.BlockSpec((BLK, NC), lambda i: (i, 0)),
            pl.BlockSpec((1, H), lambda i: (0, 0)),
            pl.BlockSpec((H, H), lambda i: (0, 0)),
            pl.BlockSpec((H, 1), lambda i: (0, 0)),
            pl.BlockSpec((1, 1), lambda i: (0, 0)),
        ],
        out_specs=pl.BlockSpec((BLK, 1), lambda i: (i, 0)),
        out_shape=jax.ShapeDtypeStruct((NP, 1), jnp.float32),
    )(p2, y1, degt, b2, w2, wd, bd)


def kernel(x, edge_index, W1, b1, W2, b2, Wd, bd):
    N = x.shape[0]
    E = edge_index.shape[1]
    NP = -(-(N + 8) // (NS * 128)) * NS * 128
    EP = -(-E // (NW * CHUNK)) * NW * CHUNK
    NCH = EP // (NW * CHUNK)

    src = edge_index[0].astype(jnp.int32)
    dst = edge_index[1].astype(jnp.int32)
    pad = jnp.full((EP - E,), N, jnp.int32)
    srcp = jnp.concatenate([src, pad]).reshape(NW, NCH, CHUNK)
    dstp = jnp.concatenate([dst, pad]).reshape(NW, NCH, CHUNK)
    x_pad = jnp.zeros((NP, D), jnp.float32).at[:N].set(x)

    degp = _make_deg(NP, NCH)(dstp)
    degt = degp.T

    y0a, y0b = _tc0(x_pad, degt, NP)
    prop64 = _make_prop(NP, NCH, D // 2)
    p1a = prop64(y0a, srcp, dstp)
    p1b = prop64(y0b, srcp, dstp)
    y1 = _tc1(p1a, p1b, y0a, y0b, degt, b1.reshape(1, H), W1, NP, N)
    p2 = _make_prop(NP, NCH, H)(y1, srcp, dstp)
    out = _tc2(p2, y1, degt, b2.reshape(1, H), W2, Wd, bd.reshape(1, 1), NP)
    return out[:N]


# speedup vs baseline: 24.7134x; 1.1849x over previous
import functools

import jax
import jax.numpy as jnp
from jax import lax
from jax.experimental import pallas as pl
from jax.experimental.pallas import tpu as pltpu
from jax.experimental.pallas import tpu_sc as plsc

NC = 2
NS = 16
NW = NC * NS
CHUNK = 128
BLK = 512
D = 128
H = 16
PKB = 14


def _sc_mesh():
    return plsc.VectorSubcoreMesh(core_axis_name="c", subcore_axis_name="s",
                                  num_cores=NC, num_subcores=NS)


def _zero_acc(zbuf, acc, s, rps, F):
    def zb(i, carry):
        for k in range(F // 16):
            zbuf[i, pl.ds(k * 16, 16)] = jnp.zeros((16,), jnp.float32)
        return carry

    lax.fori_loop(0, 128, zb, 0)
    for k in range(rps // 128):
        pltpu.sync_copy(zbuf, acc.at[pl.ds(s * rps + k * 128, 128)])


def _unpack(pk, idx_s, idx_d, nch, src_off):
    def row(i, carry):
        for k in range(CHUNK // 16):
            v = pk[i, pl.ds(k * 16, 16)]
            idx_s[i, pl.ds(k * 16, 16)] = (v & ((1 << PKB) - 1)) + src_off
            idx_d[i, pl.ds(k * 16, 16)] = lax.shift_right_logical(v, PKB)
        return carry

    lax.fori_loop(0, nch, row, 0)


def _pipelined_scatter(y_hbm, idx_s, idx_d, rows0, rows1, acc, sem0, sem1, nch):
    pltpu.async_copy(y_hbm.at[idx_s.at[0]], rows0, sem0)

    def pair(i, carry):
        j0 = 2 * i
        pltpu.async_copy(y_hbm.at[idx_s.at[j0 + 1]], rows1, sem1)
        pltpu.make_async_copy(y_hbm.at[idx_s.at[j0]], rows0, sem0).wait()
        pltpu.sync_copy(rows0, acc.at[idx_d.at[j0]], add=True)

        @pl.when(j0 + 2 < nch)
        def _():
            pltpu.async_copy(y_hbm.at[idx_s.at[j0 + 2]], rows0, sem0)

        pltpu.make_async_copy(y_hbm.at[idx_s.at[j0 + 1]], rows1, sem1).wait()
        pltpu.sync_copy(rows1, acc.at[idx_d.at[j0 + 1]], add=True)
        return carry

    lax.fori_loop(0, nch // 2, pair, 0)
    if nch % 2:
        pltpu.make_async_copy(y_hbm.at[idx_s.at[nch - 1]], rows0, sem0).wait()
        pltpu.sync_copy(rows0, acc.at[idx_d.at[nch - 1]], add=True)


@functools.lru_cache(maxsize=None)
def _make_prop1(NP, NCH2):
    F = D // 2
    rps = NP // NS

    @functools.partial(
        pl.kernel,
        out_type=jax.ShapeDtypeStruct((NC, NP, F), jnp.float32),
        mesh=_sc_mesh(),
        compiler_params=pltpu.CompilerParams(use_tc_tiling_on_sc=False),
        scratch_types=[
            pltpu.VMEM((NCH2, CHUNK), jnp.int32),
            pltpu.VMEM((NCH2, CHUNK), jnp.int32),
            pltpu.VMEM((NCH2, CHUNK), jnp.int32),
            pltpu.VMEM((CHUNK, F), jnp.float32),
            pltpu.VMEM((CHUNK, F), jnp.float32),
            pltpu.VMEM((128, F), jnp.float32),
            pltpu.VMEM_SHARED((NP, F), jnp.float32),
            pltpu.SemaphoreType.DMA,
            pltpu.SemaphoreType.DMA,
        ],
    )
    def prop(y_hbm, pk_hbm, out_hbm, pk, idx_s, idx_d, rows0, rows1, zbuf,
             acc, sem0, sem1):
        c = lax.axis_index("c")
        s = lax.axis_index("s")
        pltpu.sync_copy(pk_hbm.at[s], pk)
        _unpack(pk, idx_s, idx_d, NCH2, c * NP)
        _zero_acc(zbuf, acc, s, rps, F)
        plsc.subcore_barrier()
        _pipelined_scatter(y_hbm, idx_s, idx_d, rows0, rows1, acc, sem0, sem1,
                           NCH2)
        plsc.subcore_barrier()
        pltpu.sync_copy(acc.at[pl.ds(s * rps, rps)],
                        out_hbm.at[c, pl.ds(s * rps, rps)])

    return prop


@functools.lru_cache(maxsize=None)
def _make_prop2(NP, NCH):
    F = H
    rps = NP // NS

    @functools.partial(
        pl.kernel,
        out_type=jax.ShapeDtypeStruct((NC, NP, F), jnp.float32),
        mesh=_sc_mesh(),
        compiler_params=pltpu.CompilerParams(use_tc_tiling_on_sc=False),
        scratch_types=[
            pltpu.VMEM((NCH, CHUNK), jnp.int32),
            pltpu.VMEM((NCH, CHUNK), jnp.int32),
            pltpu.VMEM((NCH, CHUNK), jnp.int32),
            pltpu.VMEM((CHUNK, F), jnp.float32),
            pltpu.VMEM((CHUNK, F), jnp.float32),
            pltpu.VMEM((128, F), jnp.float32),
            pltpu.VMEM_SHARED((NP, F), jnp.float32),
            pltpu.SemaphoreType.DMA,
            pltpu.SemaphoreType.DMA,
        ],
    )
    def prop(y_hbm, pk_hbm, out_hbm, pk, idx_s, idx_d, rows0, rows1, zbuf,
             acc, sem0, sem1):
        c = lax.axis_index("c")
        s = lax.axis_index("s")
        wid = c * NS + s
        pltpu.sync_copy(pk_hbm.at[wid], pk)
        _unpack(pk, idx_s, idx_d, NCH, 0)
        _zero_acc(zbuf, acc, s, rps, F)
        plsc.subcore_barrier()
        _pipelined_scatter(y_hbm, idx_s, idx_d, rows0, rows1, acc, sem0, sem1,
                           NCH)
        plsc.subcore_barrier()
        pltpu.sync_copy(acc.at[pl.ds(s * rps, rps)],
                        out_hbm.at[c, pl.ds(s * rps, rps)])

    return prop


@functools.lru_cache(maxsize=None)
def _make_deg(NP, NCH):
    rps = NP // NS

    @functools.partial(
        pl.kernel,
        out_type=jax.ShapeDtypeStruct((NC, NP), jnp.float32),
        mesh=_sc_mesh(),
        compiler_params=pltpu.CompilerParams(use_tc_tiling_on_sc=False),
        scratch_types=[
            pltpu.VMEM((NCH, CHUNK), jnp.int32),
            pltpu.VMEM((NCH, CHUNK), jnp.int32),
            pltpu.VMEM((CHUNK,), jnp.float32),
            pltpu.VMEM((rps,), jnp.float32),
            pltpu.VMEM_SHARED((NP,), jnp.float32),
        ],
    )
    def deg(pk_hbm, out_hbm, pk, idx_d, ones, zbuf, acc):
        c = lax.axis_index("c")
        s = lax.axis_index("s")
        wid = c * NS + s
        pltpu.sync_copy(pk_hbm.at[wid], pk)

        def row(i, carry):
            for k in range(CHUNK // 16):
                v = pk[i, pl.ds(k * 16, 16)]
                idx_d[i, pl.ds(k * 16, 16)] = lax.shift_right_logical(v, PKB)
            return carry

        lax.fori_loop(0, NCH, row, 0)
        for i in range(CHUNK // 16):
            ones[pl.ds(i * 16, 16)] = jnp.ones((16,), jnp.float32)

        def zb(i, carry):
            zbuf[pl.ds(i * 16, 16)] = jnp.zeros((16,), jnp.float32)
            return carry

        lax.fori_loop(0, rps // 16, zb, 0)
        pltpu.sync_copy(zbuf, acc.at[pl.ds(s * rps, rps)])
        plsc.subcore_barrier()

        def step(j, carry):
            pltpu.sync_copy(ones, acc.at[idx_d.at[j]], add=True)
            return carry

        lax.fori_loop(0, NCH, step, 0)
        plsc.subcore_barrier()
        pltpu.sync_copy(acc.at[pl.ds(s * rps, rps)],
                        out_hbm.at[c, pl.ds(s * rps, rps)])

    return deg


def _dinv(deg_ref):
    d = deg_ref[:, 0:1] + deg_ref[:, 1:2] + 1.0
    return lax.rsqrt(jnp.maximum(d, 1.0))


def _tc0_body(x_ref, deg_ref, y_ref):
    y = x_ref[...] * _dinv(deg_ref)
    y_ref[0] = y[:, : D // 2]
    y_ref[1] = y[:, D // 2:]


def _tc0(x_pad, degt, NP):
    return pl.pallas_call(
        _tc0_body,
        grid=(NP // BLK,),
        in_specs=[
            pl.BlockSpec((BLK, D), lambda i: (i, 0)),
            pl.BlockSpec((BLK, NC), lambda i: (i, 0)),
        ],
        out_specs=pl.BlockSpec((NC, BLK, D // 2), lambda i: (0, i, 0)),
        out_shape=jax.ShapeDtypeStruct((NC, NP, D // 2), jnp.float32),
    )(x_pad, degt)


def _make_tc1_body(N):
    def body(p_ref, y_ref, deg_ref, b_ref, wa_ref, wb_ref, y1_ref):
        i = pl.program_id(0)
        dinv = _dinv(deg_ref)
        agg_a = (p_ref[0] + y_ref[0]) * dinv
        agg_b = (p_ref[1] + y_ref[1]) * dinv
        aw = jnp.dot(agg_a, wa_ref[...]) + jnp.dot(agg_b, wb_ref[...])
        h1 = jnp.maximum(aw + b_ref[...], 0.0)
        y1 = h1 * dinv
        row = lax.broadcasted_iota(jnp.int32, (BLK, 1), 0) + i * BLK
        y1_ref[...] = jnp.where(row < N, y1, 0.0)

    return body


def _tc1(p1, y0, degt, b1, w1, NP, N):
    return pl.pallas_call(
        _make_tc1_body(N),
        grid=(NP // BLK,),
        in_specs=[
            pl.BlockSpec((NC, BLK, D // 2), lambda i: (0, i, 0)),
            pl.BlockSpec((NC, BLK, D // 2), lambda i: (0, i, 0)),
            pl.BlockSpec((BLK, NC), lambda i: (i, 0)),
            pl.BlockSpec((1, H), lambda i: (0, 0)),
            pl.BlockSpec((D // 2, H), lambda i: (0, 0)),
            pl.BlockSpec((D // 2, H), lambda i: (0, 0)),
        ],
        out_specs=pl.BlockSpec((BLK, H), lambda i: (i, 0)),
        out_shape=jax.ShapeDtypeStruct((NP, H), jnp.float32),
    )(p1, y0, degt, b1, w1[: D // 2], w1[D // 2:])


def _tc2_body(p_ref, y1_ref, deg_ref, b_ref, w_ref, wd_ref, bd_ref, o_ref):
    dinv = _dinv(deg_ref)
    agg = (p_ref[0] + p_ref[1] + y1_ref[...]) * dinv
    h2 = jnp.maximum(jnp.dot(agg, w_ref[...]) + b_ref[...], 0.0)
    o_ref[...] = jnp.dot(h2, wd_ref[...]) + bd_ref[...]


def _tc2(p2, y1, degt, b2, w2, wd, bd, NP):
    return pl.pallas_call(
        _tc2_body,
        grid=(NP // BLK,),
        in_specs=[
            pl.BlockSpec((NC, BLK, H), lambda i: (0, i, 0)),
            pl.BlockSpec((BLK, H), lambda i: (i, 0)),
            pl.BlockSpec((BLK, NC), lambda i: (i, 0)),
            pl.BlockSpec((1, H), lambda i: (0, 0)),
            pl.BlockSpec((H, H), lambda i: (0, 0)),
            pl.BlockSpec((H, 1), lambda i: (0, 0)),
            pl.BlockSpec((1, 1), lambda i: (0, 0)),
        ],
        out_specs=pl.BlockSpec((BLK, 1), lambda i: (i, 0)),
        out_shape=jax.ShapeDtypeStruct((NP, 1), jnp.float32),
    )(p2, y1, degt, b2, w2, wd, bd)


def kernel(x, edge_index, W1, b1, W2, b2, Wd, bd):
    N = x.shape[0]
    E = edge_index.shape[1]
    assert N + 1 <= (1 << PKB)
    NP = -(-(N + 8) // (NS * 128)) * NS * 128
    EP = -(-E // (NW * CHUNK)) * NW * CHUNK
    NCH = EP // (NW * CHUNK)
    NCH2 = EP // (NS * CHUNK)

    src = edge_index[0].astype(jnp.int32)
    dst = edge_index[1].astype(jnp.int32)
    packed = src + (dst << PKB)
    pad = jnp.full((EP - E,), N + (N << PKB), jnp.int32)
    packed = jnp.concatenate([packed, pad])
    pk32 = packed.reshape(NW, NCH, CHUNK)
    pk16 = packed.reshape(NS, NCH2, CHUNK)
    x_pad = jnp.zeros((NP, D), jnp.float32).at[:N].set(x)

    degp = _make_deg(NP, NCH)(pk32)
    degt = degp.T

    y0 = _tc0(x_pad, degt, NP)
    p1 = _make_prop1(NP, NCH2)(y0.reshape(NC * NP, D // 2), pk16)
    y1 = _tc1(p1, y0, degt, b1.reshape(1, H), W1, NP, N)
    p2 = _make_prop2(NP, NCH)(y1, pk32)
    out = _tc2(p2, y1, degt, b2.reshape(1, H), W2, Wd, bd.reshape(1, 1), NP)
    return out[:N]
